# baseline (speedup 1.0000x reference)
import functools

import jax
import jax.numpy as jnp
from jax import lax
from jax.experimental import pallas as pl
from jax.experimental.pallas import tpu as pltpu
from jax.experimental.pallas import tpu_sc as plsc

_N = 10000
_E = 320000
_D = 128

_NC = 2
_NS = 16
_NW = _NC * _NS
_CHUNK = 128
_NB = 2
_CPT = 80
_NG = _CPT // _NB
_EPAD = _NW * _CHUNK * _CPT
_NPAD = 10112
_STRIPE = _NPAD // _NS

@functools.cache
def _mesh():
    return plsc.VectorSubcoreMesh(core_axis_name="c", subcore_axis_name="s",
                                  num_cores=_NC, num_subcores=_NS)


def _sc_agg_body(x_hbm, src_hbm, dst_hbm, zrow_hbm, acc_out,
                 sring_v, dst_v, rows_v, acc_sh, is0, is1, gs0, gs1):
    isems = (is0, is1)
    gsems = (gs0, gs1)

    ci = lax.axis_index("c")
    si = lax.axis_index("s")
    g = ci * _NS + si
    r0 = si * _STRIPE

    pltpu.sync_copy(zrow_hbm, acc_sh.at[pl.ds(r0, _STRIPE)])
    pltpu.sync_copy(dst_hbm.at[g], dst_v)
    pltpu.async_copy(src_hbm.at[g, 0], sring_v.at[0], isems[0])
    pltpu.async_copy(src_hbm.at[g, 1], sring_v.at[1], isems[1])
    plsc.subcore_barrier()

    pltpu.make_async_copy(src_hbm.at[g, 0], sring_v.at[0], isems[0]).wait()
    pltpu.async_copy(x_hbm.at[sring_v.at[0]], rows_v.at[0], gsems[0])

    def outer(j2, carry):
        for p in range(2):
            j = 2 * j2 + p
            pltpu.make_async_copy(x_hbm.at[sring_v.at[p]], rows_v.at[p],
                                  gsems[p]).wait()
            pltpu.async_copy(src_hbm.at[g, j + 2], sring_v.at[p], isems[p])
            pltpu.make_async_copy(src_hbm.at[g, j + 1], sring_v.at[1 - p],
                                  isems[1 - p]).wait()
            pltpu.async_copy(x_hbm.at[sring_v.at[1 - p]], rows_v.at[1 - p],
                             gsems[1 - p])
            pltpu.sync_copy(rows_v.at[p], acc_sh.at[dst_v.at[j]], add=True)
        return carry

    lax.fori_loop(0, _CPT // 2, outer, 0)
    pltpu.make_async_copy(x_hbm.at[sring_v.at[0]], rows_v.at[0],
                          gsems[0]).wait()
    pltpu.make_async_copy(src_hbm.at[g, 1], sring_v.at[1], isems[1]).wait()

    plsc.subcore_barrier()

    pltpu.sync_copy(acc_sh.at[pl.ds(r0, _STRIPE)],
                    acc_out.at[ci, pl.ds(r0, _STRIPE)])


def _make_sc_agg():
    return pl.kernel(
        _sc_agg_body,
        out_type=jax.ShapeDtypeStruct((_NC, _NPAD, _D), jnp.float32),
        mesh=_mesh(),
        scratch_types=[
            pltpu.VMEM((2, _CHUNK), jnp.int32),
            pltpu.VMEM((_CPT, _CHUNK), jnp.int32),
            pltpu.VMEM((2, _CHUNK, _D), jnp.float32),
            pltpu.VMEM_SHARED((_NPAD, _D), jnp.float32),
        ] + [pltpu.SemaphoreType.DMA] * 4,
    )


def _sc_cnt_body(dst_hbm, zvec_hbm, cnt_out, dst_v, hist_v):
    ci = lax.axis_index("c")
    si = lax.axis_index("s")
    g = ci * _NS + si

    pltpu.sync_copy(zvec_hbm, hist_v)
    pltpu.sync_copy(dst_hbm.at[g], dst_v)

    ones16 = jnp.ones((16,), jnp.float32)

    def hstep(j, carry):
        def inner(k, c2):
            vec = dst_v[j, pl.ds(k * 16, 16)]
            plsc.addupdate_scatter(hist_v, [vec], ones16)
            return c2
        return lax.fori_loop(0, _CHUNK // 16, inner, carry)

    lax.fori_loop(0, _CPT, hstep, 0)
    pltpu.sync_copy(hist_v, cnt_out.at[g])


def _make_sc_cnt():
    return pl.kernel(
        _sc_cnt_body,
        out_type=jax.ShapeDtypeStruct((_NW, _NPAD), jnp.float32),
        mesh=_mesh(),
        scratch_types=[
            pltpu.VMEM((_CPT, _CHUNK), jnp.int32),
            pltpu.VMEM((_NPAD,), jnp.float32),
        ],
        compiler_params=pltpu.CompilerParams(needs_layout_passes=False),
    )


def _dense1(accP, cntP, x, w1l, b1l, w1r, w3, b3):
    blk = 1000

    def body(accP_ref, cntP_ref, x_ref, w1l_ref, b1l_ref, w1r_ref, w3_ref,
             b3_ref, o_ref):
        acc = accP_ref[0] + accP_ref[1]
        cnt = jnp.sum(cntP_ref[...], axis=0)
        mean = acc / jnp.maximum(cnt, 1.0)
        h1 = jnp.maximum(
            jnp.dot(mean, w1l_ref[...], preferred_element_type=jnp.float32)
            + b1l_ref[...]
            + jnp.dot(x_ref[...], w1r_ref[...], preferred_element_type=jnp.float32),
            0.0)
        o_ref[...] = jnp.maximum(
            jnp.dot(h1, w3_ref[...], preferred_element_type=jnp.float32)
            + b3_ref[...], 0.0)

    return pl.pallas_call(
        body,
        grid=(_N // blk,),
        in_specs=[
            pl.BlockSpec((2, blk, _D), lambda i: (0, i, 0)),
            pl.BlockSpec((_NW, blk, 1), lambda i: (0, i, 0)),
            pl.BlockSpec((blk, _D), lambda i: (i, 0)),
            pl.BlockSpec((_D, _D), lambda i: (0, 0)),
            pl.BlockSpec((1, _D), lambda i: (0, 0)),
            pl.BlockSpec((_D, _D), lambda i: (0, 0)),
            pl.BlockSpec((_D, _D), lambda i: (0, 0)),
            pl.BlockSpec((1, _D), lambda i: (0, 0)),
        ],
        out_specs=pl.BlockSpec((blk, _D), lambda i: (i, 0)),
        out_shape=jax.ShapeDtypeStruct((_N, _D), jnp.float32),
    )(accP, cntP, x, w1l, b1l, w1r, w3, b3)


def _dense2(accP, cntP, h, w2l, b2l, w2r, w4, b4):
    blk = 1000

    def body(accP_ref, cntP_ref, h_ref, w2l_ref, b2l_ref, w2r_ref, w4_ref,
             b4_ref, o_ref):
        acc = accP_ref[0] + accP_ref[1]
        cnt = jnp.sum(cntP_ref[...], axis=0)
        mean = acc / jnp.maximum(cnt, 1.0)
        h3 = jnp.maximum(
            jnp.dot(mean, w2l_ref[...], preferred_element_type=jnp.float32)
            + b2l_ref[...]
            + jnp.dot(h_ref[...], w2r_ref[...], preferred_element_type=jnp.float32),
            0.0)
        o_ref[...] = (jnp.dot(h3, w4_ref[...], preferred_element_type=jnp.float32)
                      + b4_ref[...])

    return pl.pallas_call(
        body,
        grid=(_N // blk,),
        in_specs=[
            pl.BlockSpec((2, blk, _D), lambda i: (0, i, 0)),
            pl.BlockSpec((_NW, blk, 1), lambda i: (0, i, 0)),
            pl.BlockSpec((blk, _D), lambda i: (i, 0)),
            pl.BlockSpec((_D, _D), lambda i: (0, 0)),
            pl.BlockSpec((1, _D), lambda i: (0, 0)),
            pl.BlockSpec((_D, _D), lambda i: (0, 0)),
            pl.BlockSpec((_D, _D), lambda i: (0, 0)),
            pl.BlockSpec((1, _D), lambda i: (0, 0)),
        ],
        out_specs=pl.BlockSpec((blk, _D), lambda i: (i, 0)),
        out_shape=jax.ShapeDtypeStruct((_N, _D), jnp.float32),
    )(accP, cntP, h, w2l, b2l, w2r, w4, b4)


def kernel(x, edge_index, W1_l, b1_l, W1_r, lin1_W, lin1_b, W2_l, b2_l, W2_r,
           lin2_W, lin2_b):
    src = edge_index[0]
    dst = edge_index[1]
    pad = _EPAD - _E
    src_p = jnp.concatenate([src, jnp.zeros((pad,), jnp.int32)])
    dst_p = jnp.concatenate([dst, jnp.full((pad,), _N, jnp.int32)])
    src_g = src_p.reshape(_NW, _CPT, _CHUNK)
    src_g = jnp.concatenate(
        [src_g, jnp.zeros((_NW, 2, _CHUNK), jnp.int32)], axis=1)
    dst_p = dst_p.reshape(_NW, _CPT, _CHUNK)

    zrow = jnp.zeros((_STRIPE, _D), jnp.float32)
    zvec = jnp.zeros((_NPAD,), jnp.float32)

    accP = _make_sc_agg()(x, src_g, dst_p, zrow)
    cntV = _make_sc_cnt()(dst_p, zvec)
    cntP = cntV.reshape(_NW, _NPAD, 1)
    h = _dense1(accP, cntP, x,
                W1_l.T, b1_l.reshape(1, _D), W1_r.T,
                lin1_W.T, lin1_b.reshape(1, _D))
    accP2 = _make_sc_agg()(h, src_g, dst_p, zrow)
    out = _dense2(accP2, cntP, h,
                  W2_l.T, b2_l.reshape(1, _D), W2_r.T,
                  lin2_W.T, lin2_b.reshape(1, _D))
    return out


# speedup vs baseline: 1.7336x; 1.7336x over previous
import functools

import jax
import jax.numpy as jnp
from jax import lax
from jax.experimental import pallas as pl
from jax.experimental.pallas import tpu as pltpu
from jax.experimental.pallas import tpu_sc as plsc

_N = 10000
_E = 320000
_D = 128

_NC = 2
_NS = 16
_NW = _NC * _NS
_CHUNK = 128
_NB = 2
_CPT = 79
_NG = _CPT // _NB
_EPAD = _NW * _CHUNK * _CPT
_NPAD = 10112
_STRIPE = _NPAD // _NS

@functools.cache
def _mesh():
    return plsc.VectorSubcoreMesh(core_axis_name="c", subcore_axis_name="s",
                                  num_cores=_NC, num_subcores=_NS)


def _sc_agg_body(with_cnt, *refs):
    if with_cnt:
        (x_hbm, src_hbm, dst_hbm, zrow_hbm, zvec_hbm,
         acc_out, cnt_out, src_v, dst_v, rows_v, acc_sh, hist_v) = refs
    else:
        (x_hbm, src_hbm, dst_hbm, zrow_hbm,
         acc_out, src_v, dst_v, rows_v, acc_sh) = refs
    ci = lax.axis_index("c")
    si = lax.axis_index("s")
    g = ci * _NS + si
    r0 = si * _STRIPE

    pltpu.sync_copy(zrow_hbm, acc_sh.at[pl.ds(r0, _STRIPE)])
    pltpu.sync_copy(src_hbm.at[g], src_v)
    pltpu.sync_copy(dst_hbm.at[g], dst_v)
    if with_cnt:
        pltpu.sync_copy(zvec_hbm, hist_v)
    plsc.subcore_barrier()

    if with_cnt:
        ones16 = jnp.ones((16,), jnp.float32)

        def hstep(j, carry):
            def inner(k, c2):
                vec = dst_v[j, pl.ds(k * 16, 16)]
                plsc.addupdate_scatter(hist_v, [vec], ones16)
                return c2
            return lax.fori_loop(0, _CHUNK // 16, inner, carry)

        lax.fori_loop(0, _CPT, hstep, 0)

    def step(j, carry):
        pltpu.sync_copy(x_hbm.at[src_v.at[j]], rows_v)
        pltpu.sync_copy(rows_v, acc_sh.at[dst_v.at[j]], add=True)
        return carry

    lax.fori_loop(0, _CPT, step, 0)
    plsc.subcore_barrier()

    pltpu.sync_copy(acc_sh.at[pl.ds(r0, _STRIPE)],
                    acc_out.at[ci, pl.ds(r0, _STRIPE)])
    if with_cnt:
        pltpu.sync_copy(hist_v, cnt_out.at[g])


def _make_sc_agg(with_cnt):
    out_type = [jax.ShapeDtypeStruct((_NC, _NPAD, _D), jnp.float32)]
    scratch = [
        pltpu.VMEM((_CPT, _CHUNK), jnp.int32),
        pltpu.VMEM((_CPT, _CHUNK), jnp.int32),
        pltpu.VMEM((_CHUNK, _D), jnp.float32),
        pltpu.VMEM_SHARED((_NPAD, _D), jnp.float32),
    ]
    params = None
    if with_cnt:
        out_type.append(jax.ShapeDtypeStruct((_NW, _NPAD), jnp.float32))
        scratch.append(pltpu.VMEM((_NPAD,), jnp.float32))
        params = pltpu.CompilerParams(needs_layout_passes=False)
    return pl.kernel(
        functools.partial(_sc_agg_body, with_cnt),
        out_type=tuple(out_type) if with_cnt else out_type[0],
        mesh=_mesh(),
        scratch_types=scratch,
        compiler_params=params,
    )


def _dense1(accP, cntP, x, w1l, b1l, w1r, w3, b3):
    blk = 1000

    def body(accP_ref, cntP_ref, x_ref, w1l_ref, b1l_ref, w1r_ref, w3_ref,
             b3_ref, o_ref):
        acc = accP_ref[0] + accP_ref[1]
        cnt = jnp.sum(cntP_ref[...], axis=0)
        mean = acc / jnp.maximum(cnt, 1.0)
        h1 = jnp.maximum(
            jnp.dot(mean, w1l_ref[...], preferred_element_type=jnp.float32)
            + b1l_ref[...]
            + jnp.dot(x_ref[...], w1r_ref[...], preferred_element_type=jnp.float32),
            0.0)
        o_ref[...] = jnp.maximum(
            jnp.dot(h1, w3_ref[...], preferred_element_type=jnp.float32)
            + b3_ref[...], 0.0)

    return pl.pallas_call(
        body,
        grid=(_N // blk,),
        in_specs=[
            pl.BlockSpec((2, blk, _D), lambda i: (0, i, 0)),
            pl.BlockSpec((_NW, blk, 1), lambda i: (0, i, 0)),
            pl.BlockSpec((blk, _D), lambda i: (i, 0)),
            pl.BlockSpec((_D, _D), lambda i: (0, 0)),
            pl.BlockSpec((1, _D), lambda i: (0, 0)),
            pl.BlockSpec((_D, _D), lambda i: (0, 0)),
            pl.BlockSpec((_D, _D), lambda i: (0, 0)),
            pl.BlockSpec((1, _D), lambda i: (0, 0)),
        ],
        out_specs=pl.BlockSpec((blk, _D), lambda i: (i, 0)),
        out_shape=jax.ShapeDtypeStruct((_N, _D), jnp.float32),
    )(accP, cntP, x, w1l, b1l, w1r, w3, b3)


def _dense2(accP, cntP, h, w2l, b2l, w2r, w4, b4):
    blk = 1000

    def body(accP_ref, cntP_ref, h_ref, w2l_ref, b2l_ref, w2r_ref, w4_ref,
             b4_ref, o_ref):
        acc = accP_ref[0] + accP_ref[1]
        cnt = jnp.sum(cntP_ref[...], axis=0)
        mean = acc / jnp.maximum(cnt, 1.0)
        h3 = jnp.maximum(
            jnp.dot(mean, w2l_ref[...], preferred_element_type=jnp.float32)
            + b2l_ref[...]
            + jnp.dot(h_ref[...], w2r_ref[...], preferred_element_type=jnp.float32),
            0.0)
        o_ref[...] = (jnp.dot(h3, w4_ref[...], preferred_element_type=jnp.float32)
                      + b4_ref[...])

    return pl.pallas_call(
        body,
        grid=(_N // blk,),
        in_specs=[
            pl.BlockSpec((2, blk, _D), lambda i: (0, i, 0)),
            pl.BlockSpec((_NW, blk, 1), lambda i: (0, i, 0)),
            pl.BlockSpec((blk, _D), lambda i: (i, 0)),
            pl.BlockSpec((_D, _D), lambda i: (0, 0)),
            pl.BlockSpec((1, _D), lambda i: (0, 0)),
            pl.BlockSpec((_D, _D), lambda i: (0, 0)),
            pl.BlockSpec((_D, _D), lambda i: (0, 0)),
            pl.BlockSpec((1, _D), lambda i: (0, 0)),
        ],
        out_specs=pl.BlockSpec((blk, _D), lambda i: (i, 0)),
        out_shape=jax.ShapeDtypeStruct((_N, _D), jnp.float32),
    )(accP, cntP, h, w2l, b2l, w2r, w4, b4)


def kernel(x, edge_index, W1_l, b1_l, W1_r, lin1_W, lin1_b, W2_l, b2_l, W2_r,
           lin2_W, lin2_b):
    src = edge_index[0]
    dst = edge_index[1]
    ppt = _CPT * _CHUNK - _E // _NW
    pad_dst = jnp.broadcast_to(_N + (jnp.arange(ppt, dtype=jnp.int32)
                                     % (_NPAD - _N)), (_NW, ppt))
    src_p = jnp.concatenate(
        [src.reshape(_NW, _E // _NW),
         jnp.zeros((_NW, ppt), jnp.int32)], axis=1)
    dst_p = jnp.concatenate(
        [dst.reshape(_NW, _E // _NW), pad_dst], axis=1)
    src_p = src_p.reshape(_NW, _CPT, _CHUNK)
    dst_p = dst_p.reshape(_NW, _CPT, _CHUNK)

    zrow = jnp.zeros((_STRIPE, _D), jnp.float32)
    zvec = jnp.zeros((_NPAD,), jnp.float32)

    accP, cntV = _make_sc_agg(True)(x, src_p, dst_p, zrow, zvec)
    cntP = cntV.reshape(_NW, _NPAD, 1)
    h = _dense1(accP, cntP, x,
                W1_l.T, b1_l.reshape(1, _D), W1_r.T,
                lin1_W.T, lin1_b.reshape(1, _D))
    accP2 = _make_sc_agg(False)(h, src_p, dst_p, zrow)
    out = _dense2(accP2, cntP, h,
                  W2_l.T, b2_l.reshape(1, _D), W2_r.T,
                  lin2_W.T, lin2_b.reshape(1, _D))
    return out
